# baseline (speedup 1.0000x reference)
import functools

import jax
import jax.numpy as jnp
from jax import lax
from jax.experimental import pallas as pl
from jax.experimental.pallas import tpu as pltpu
from jax.experimental.pallas import tpu_sc as plsc

N_ROWS = 320000
D = 128
N_SEG = 10000
EPS = 0.001

BR1 = 6400
NB1 = N_ROWS // BR1


def _stats_body(a_ref, w_ref, g_ref, bt_ref, wsc_ref, bcol_ref, m_acc, cs_acc):
    i = pl.program_id(0)

    @pl.when(i == 0)
    def _():
        m_acc[...] = jnp.zeros_like(m_acc)
        cs_acc[...] = jnp.zeros_like(cs_acc)

    a = a_ref[...]
    m_acc[...] += lax.dot_general(a, a, (((0,), (0,)), ((), ())),
                                  preferred_element_type=jnp.float32)
    cs_acc[...] += jnp.sum(a, axis=0, keepdims=True)

    @pl.when(i == NB1 - 1)
    def _():
        w = w_ref[...]
        mean_col = lax.dot_general(w, cs_acc[...], (((1,), (1,)), ((), ())),
                                   preferred_element_type=jnp.float32) / N_ROWS
        u = lax.dot_general(w, m_acc[...], (((1,), (0,)), ((), ())),
                            preferred_element_type=jnp.float32)
        var_col = jnp.sum(u * w, axis=1, keepdims=True) / N_ROWS - mean_col * mean_col
        s_col = g_ref[...] / jnp.sqrt(var_col + EPS)
        wsc_ref[...] = w * s_col
        bcol_ref[...] = bt_ref[...] - mean_col * s_col


def _stats_pass(a, w, gamma_col, beta_col):
    return pl.pallas_call(
        _stats_body,
        grid=(NB1,),
        in_specs=[
            pl.BlockSpec((BR1, D), lambda i: (i, 0)),
            pl.BlockSpec((D, D), lambda i: (0, 0)),
            pl.BlockSpec((D, 1), lambda i: (0, 0)),
            pl.BlockSpec((D, 1), lambda i: (0, 0)),
        ],
        out_specs=[
            pl.BlockSpec((D, D), lambda i: (0, 0)),
            pl.BlockSpec((D, 1), lambda i: (0, 0)),
        ],
        out_shape=[
            jax.ShapeDtypeStruct((D, D), jnp.float32),
            jax.ShapeDtypeStruct((D, 1), jnp.float32),
        ],
        scratch_shapes=[
            pltpu.VMEM((D, D), jnp.float32),
            pltpu.VMEM((1, D), jnp.float32),
        ],
    )(a, w, gamma_col, beta_col)


N_CHUNKS = 5
CHUNK = N_ROWS // N_CHUNKS
BR2 = 6400
NB2 = CHUNK // BR2


def _matmul_body(a_ref, wsc_ref, b_ref, y_ref):
    y = lax.dot_general(a_ref[...], wsc_ref[...], (((1,), (1,)), ((), ())),
                        preferred_element_type=jnp.float32)
    y_ref[...] = jnp.maximum(y + b_ref[...], 0.0)


def _matmul_pass(a_full, wsc, b_row, chunk_i):
    base = chunk_i * NB2
    return pl.pallas_call(
        _matmul_body,
        grid=(NB2,),
        in_specs=[
            pl.BlockSpec((BR2, D), lambda i: (base + i, 0)),
            pl.BlockSpec((D, D), lambda i: (0, 0)),
            pl.BlockSpec((1, D), lambda i: (0, 0)),
        ],
        out_specs=pl.BlockSpec((BR2, D), lambda i: (i, 0)),
        out_shape=jax.ShapeDtypeStruct((CHUNK, D), jnp.float32),
    )(a_full, wsc, b_row)


_SC_INFO = plsc.get_sparse_core_info()
NC = _SC_INFO.num_cores
NS = _SC_INFO.num_subcores
NW = NC * NS
ROWS_PER_W = CHUNK // NW
WIN = 128
NWIN = ROWS_PER_W // WIN
TAIL = ROWS_PER_W - NWIN * WIN
N_SEG_PAD = 10240
SEG_PER_TILE = N_SEG_PAD // NS
_PAIRS = (NWIN - 2) // 2


def _sc_segment_sum(y_chunk, idx_full, zeros, chunk_i):
    chunk_start = chunk_i * CHUNK
    mesh = plsc.VectorSubcoreMesh(core_axis_name="c", subcore_axis_name="s")

    @functools.partial(
        pl.kernel,
        out_type=jax.ShapeDtypeStruct((NC * N_SEG_PAD, D), jnp.float32),
        mesh=mesh,
        scratch_types=[
            pltpu.VMEM((WIN,), jnp.int32),
            pltpu.VMEM((WIN,), jnp.int32),
            pltpu.VMEM((WIN, D), jnp.float32),
            pltpu.VMEM((WIN, D), jnp.float32),
            pltpu.VMEM((TAIL,), jnp.int32),
            pltpu.VMEM((TAIL, D), jnp.float32),
            pltpu.VMEM_SHARED((N_SEG_PAD, D), jnp.float32),
            pltpu.SemaphoreType.DMA,
            pltpu.SemaphoreType.DMA,
            pltpu.SemaphoreType.DMA,
            pltpu.SemaphoreType.DMA,
            pltpu.SemaphoreType.DMA,
            pltpu.SemaphoreType.DMA,
        ],
    )
    def k(y_hbm, idx_hbm, zeros_hbm, out_hbm,
          idx_v0, idx_v1, y_v0, y_v1, idx_t, y_t, acc,
          sem_i0, sem_i1, sem_y0, sem_y1, sem_s0, sem_s1):
        c = lax.axis_index("c")
        s = lax.axis_index("s")
        wid = s * NC + c
        idx_bufs = (idx_v0, idx_v1)
        y_bufs = (y_v0, y_v1)
        sem_i = (sem_i0, sem_i1)
        sem_y = (sem_y0, sem_y1)
        sem_s = (sem_s0, sem_s1)
        base0 = wid * ROWS_PER_W

        def start_gather(w, b):
            base = base0 + w * WIN
            pltpu.async_copy(idx_hbm.at[pl.ds(chunk_start + base, WIN)],
                             idx_bufs[b], sem_i[b])
            pltpu.async_copy(y_hbm.at[pl.ds(base, WIN)], y_bufs[b], sem_y[b])

        def wait_gather(b):
            pltpu.make_async_copy(idx_hbm.at[pl.ds(0, WIN)], idx_bufs[b],
                                  sem_i[b]).wait()
            pltpu.make_async_copy(y_hbm.at[pl.ds(0, WIN)], y_bufs[b],
                                  sem_y[b]).wait()

        def start_scatter(b):
            pltpu.async_copy(y_bufs[b], acc.at[idx_bufs[b]], sem_s[b], add=True)

        def wait_scatter(b):
            pltpu.make_async_copy(y_bufs[b], acc.at[idx_bufs[b]],
                                  sem_s[b]).wait()

        pltpu.sync_copy(zeros_hbm.at[pl.ds(s * SEG_PER_TILE, SEG_PER_TILE)],
                        acc.at[pl.ds(s * SEG_PER_TILE, SEG_PER_TILE)])
        plsc.subcore_barrier()

        start_gather(0, 0)
        start_gather(1, 1)

        def body(i, carry):
            w = i * 2
            for b in range(2):
                wait_gather(b)
                start_scatter(b)
                wait_scatter(b)
                start_gather(w + b + 2, b)
            return carry

        lax.fori_loop(0, _PAIRS, body, 0)
        for w in range(2 * _PAIRS, NWIN):
            b = w % 2
            wait_gather(b)
            start_scatter(b)
            wait_scatter(b)
            if w + 2 < NWIN:
                start_gather(w + 2, b)

        tbase = base0 + NWIN * WIN
        pltpu.sync_copy(idx_hbm.at[pl.ds(chunk_start + tbase, TAIL)], idx_t)
        pltpu.sync_copy(y_hbm.at[pl.ds(tbase, TAIL)], y_t)
        pltpu.sync_copy(y_t, acc.at[idx_t], add=True)

        plsc.subcore_barrier()
        pltpu.sync_copy(
            acc.at[pl.ds(s * SEG_PER_TILE, SEG_PER_TILE)],
            out_hbm.at[pl.ds(c * N_SEG_PAD + s * SEG_PER_TILE, SEG_PER_TILE)])

    return k(y_chunk, idx_full, zeros)


BR3 = 1024
NB3 = N_SEG_PAD // BR3


def _combine_body(*refs):
    o_ref = refs[-1]
    acc = None
    for p_ref in refs[:-1]:
        t = p_ref[0] + p_ref[1]
        acc = t if acc is None else acc + t
    o_ref[...] = acc


def _combine(parts):
    return pl.pallas_call(
        _combine_body,
        grid=(NB3,),
        in_specs=[pl.BlockSpec((2, BR3, D), lambda i: (0, i, 0))
                  for _ in parts],
        out_specs=pl.BlockSpec((BR3, D), lambda i: (i, 0)),
        out_shape=jax.ShapeDtypeStruct((N_SEG_PAD, D), jnp.float32),
    )(*parts)


def kernel(inputs, unq_inv, W, gamma, beta):
    idx = unq_inv.astype(jnp.int32)
    gamma_col = gamma.reshape(D, 1)
    beta_col = beta.reshape(D, 1)
    wsc, b_col = _stats_pass(inputs, W, gamma_col, beta_col)
    b_row = b_col.reshape(1, D)
    zeros = jnp.zeros((N_SEG_PAD, D), jnp.float32)
    partials = []
    for ci in range(N_CHUNKS):
        y_chunk = _matmul_pass(inputs, wsc, b_row, ci)
        partials.append(_sc_segment_sum(y_chunk, idx, zeros, ci))
    out = _combine([p.reshape(NC, N_SEG_PAD, D) for p in partials])
    return out[:N_SEG]


# speedup vs baseline: 1.3721x; 1.3721x over previous
import functools

import jax
import jax.numpy as jnp
from jax import lax
from jax.experimental import pallas as pl
from jax.experimental.pallas import tpu as pltpu
from jax.experimental.pallas import tpu_sc as plsc

N_ROWS = 320000
D = 128
N_SEG = 10000
EPS = 0.001

BR1 = 6400
NB1 = N_ROWS // BR1


def _mm_stats_body(a_ref, w_ref, g_ref, bt_ref, y_ref, s_ref, b_ref,
                   cs_acc, ss_acc):
    i = pl.program_id(0)

    @pl.when(i == 0)
    def _():
        cs_acc[...] = jnp.zeros_like(cs_acc)
        ss_acc[...] = jnp.zeros_like(ss_acc)

    x = lax.dot_general(a_ref[...], w_ref[...], (((1,), (1,)), ((), ())),
                        preferred_element_type=jnp.float32)
    y_ref[...] = x
    cs_acc[...] += jnp.sum(x, axis=0, keepdims=True)
    ss_acc[...] += jnp.sum(x * x, axis=0, keepdims=True)

    @pl.when(i == NB1 - 1)
    def _():
        mean = cs_acc[...] / N_ROWS
        var = ss_acc[...] / N_ROWS - mean * mean
        s = g_ref[...] / jnp.sqrt(var + EPS)
        s_ref[...] = s
        b_ref[...] = bt_ref[...] - mean * s


def _mm_stats_pass(a, w, gamma_row, beta_row):
    return pl.pallas_call(
        _mm_stats_body,
        grid=(NB1,),
        in_specs=[
            pl.BlockSpec((BR1, D), lambda i: (i, 0)),
            pl.BlockSpec((D, D), lambda i: (0, 0)),
            pl.BlockSpec((1, D), lambda i: (0, 0)),
            pl.BlockSpec((1, D), lambda i: (0, 0)),
        ],
        out_specs=[
            pl.BlockSpec((BR1, D), lambda i: (i, 0)),
            pl.BlockSpec((1, D), lambda i: (0, 0)),
            pl.BlockSpec((1, D), lambda i: (0, 0)),
        ],
        out_shape=[
            jax.ShapeDtypeStruct((N_ROWS, D), jnp.float32),
            jax.ShapeDtypeStruct((1, D), jnp.float32),
            jax.ShapeDtypeStruct((1, D), jnp.float32),
        ],
        scratch_shapes=[
            pltpu.VMEM((1, D), jnp.float32),
            pltpu.VMEM((1, D), jnp.float32),
        ],
    )(a, w, gamma_row, beta_row)


_SC_INFO = plsc.get_sparse_core_info()
NC = _SC_INFO.num_cores
NS = _SC_INFO.num_subcores
NW = NC * NS
L = _SC_INFO.num_lanes
NG = D // L
ROWS_PER_W = N_ROWS // NW
WIN = 96
NWIN = ROWS_PER_W // WIN
TAIL = ROWS_PER_W - NWIN * WIN
N_SEG_PAD = 10240
SEG_PER_TILE = N_SEG_PAD // NS
NBUF = 3


def _sc_segment_sum(y, idx, zeros, s_row, b_row):
    mesh = plsc.VectorSubcoreMesh(core_axis_name="c", subcore_axis_name="s")

    scratch = (
        [pltpu.VMEM((WIN,), jnp.int32) for _ in range(NBUF)]
        + [pltpu.VMEM((WIN, D), jnp.float32) for _ in range(NBUF)]
        + [pltpu.VMEM((TAIL,), jnp.int32), pltpu.VMEM((TAIL, D), jnp.float32),
           pltpu.VMEM((D,), jnp.float32), pltpu.VMEM((D,), jnp.float32),
           pltpu.VMEM_SHARED((N_SEG_PAD, D), jnp.float32)]
        + [pltpu.SemaphoreType.DMA for _ in range(3 * NBUF + 1)]
    )

    @functools.partial(
        pl.kernel,
        out_type=jax.ShapeDtypeStruct((NC * N_SEG_PAD, D), jnp.float32),
        mesh=mesh,
        scratch_types=scratch,
    )
    def k(y_hbm, idx_hbm, zeros_hbm, s_hbm, b_hbm, out_hbm, *refs):
        idx_bufs = refs[0:NBUF]
        y_bufs = refs[NBUF:2 * NBUF]
        idx_t, y_t, s_v, b_v, acc = refs[2 * NBUF:2 * NBUF + 5]
        sems = refs[2 * NBUF + 5:]
        sem_i = sems[0:NBUF]
        sem_y = sems[NBUF:2 * NBUF]
        sem_s = sems[2 * NBUF:3 * NBUF]
        sem_misc = sems[3 * NBUF]

        c = lax.axis_index("c")
        s = lax.axis_index("s")
        wid = s * NC + c
        base0 = wid * ROWS_PER_W

        def start_gather(w, b):
            base = base0 + w * WIN
            pltpu.async_copy(idx_hbm.at[pl.ds(base, WIN)], idx_bufs[b], sem_i[b])
            pltpu.async_copy(y_hbm.at[pl.ds(base, WIN)], y_bufs[b], sem_y[b])

        def wait_gather(b):
            pltpu.make_async_copy(idx_hbm.at[pl.ds(0, WIN)], idx_bufs[b],
                                  sem_i[b]).wait()
            pltpu.make_async_copy(y_hbm.at[pl.ds(0, WIN)], y_bufs[b],
                                  sem_y[b]).wait()

        def start_scatter(b):
            pltpu.async_copy(y_bufs[b], acc.at[idx_bufs[b]], sem_s[b], add=True)

        def wait_scatter(b):
            pltpu.make_async_copy(y_bufs[b], acc.at[idx_bufs[b]],
                                  sem_s[b]).wait()

        pltpu.sync_copy(s_hbm, s_v)
        pltpu.sync_copy(b_hbm, b_v)
        s_regs = [s_v[pl.ds(j * L, L)] for j in range(NG)]
        b_regs = [b_v[pl.ds(j * L, L)] for j in range(NG)]

        def affine(b):
            yb = y_bufs[b]

            def row(r, carry):
                for j in range(NG):
                    v = yb[r, pl.ds(j * L, L)]
                    yb[r, pl.ds(j * L, L)] = jnp.maximum(
                        v * s_regs[j] + b_regs[j], 0.0)
                return carry

            lax.fori_loop(0, WIN, row, 0)

        pltpu.sync_copy(zeros_hbm.at[pl.ds(s * SEG_PER_TILE, SEG_PER_TILE)],
                        acc.at[pl.ds(s * SEG_PER_TILE, SEG_PER_TILE)])
        plsc.subcore_barrier()

        start_gather(0, 0)
        start_gather(1, 1)
        wait_gather(0)
        affine(0)
        start_scatter(0)
        start_gather(2, 2)

        def body3(i, carry):
            w0 = i * NBUF + 1
            for u in range(NBUF):
                w = w0 + u
                b = (1 + u) % NBUF
                bb = u
                wait_gather(b)
                affine(b)
                start_scatter(b)
                wait_scatter(bb)
                start_gather(w + 2, bb)
            return carry

        n_triples = (NWIN - 3) // NBUF
        lax.fori_loop(0, n_triples, body3, 0)
        for w in range(n_triples * NBUF + 1, NWIN):
            b = w % NBUF
            wait_gather(b)
            affine(b)
            start_scatter(b)
            if w + 2 < NWIN:
                bb = (w + 2) % NBUF
                wait_scatter(bb)
                start_gather(w + 2, bb)

        tbase = base0 + NWIN * WIN
        pltpu.sync_copy(idx_hbm.at[pl.ds(tbase, TAIL)], idx_t)
        pltpu.sync_copy(y_hbm.at[pl.ds(tbase, TAIL)], y_t)

        def trow(r, carry):
            for j in range(NG):
                v = y_t[r, pl.ds(j * L, L)]
                y_t[r, pl.ds(j * L, L)] = jnp.maximum(
                    v * s_regs[j] + b_regs[j], 0.0)
            return carry

        lax.fori_loop(0, TAIL, trow, 0)
        pltpu.sync_copy(y_t, acc.at[idx_t], add=True)

        for b in range(NBUF):
            wait_scatter(b)

        plsc.subcore_barrier()
        pltpu.sync_copy(
            acc.at[pl.ds(s * SEG_PER_TILE, SEG_PER_TILE)],
            out_hbm.at[pl.ds(c * N_SEG_PAD + s * SEG_PER_TILE, SEG_PER_TILE)])

    return k(y, idx, zeros, s_row, b_row)


BR3 = 1024
NB3 = N_SEG_PAD // BR3


def _combine_body(p_ref, o_ref):
    o_ref[...] = p_ref[0] + p_ref[1]


def _combine(p):
    return pl.pallas_call(
        _combine_body,
        grid=(NB3,),
        in_specs=[pl.BlockSpec((2, BR3, D), lambda i: (0, i, 0))],
        out_specs=pl.BlockSpec((BR3, D), lambda i: (i, 0)),
        out_shape=jax.ShapeDtypeStruct((N_SEG_PAD, D), jnp.float32),
    )(p)


def kernel(inputs, unq_inv, W, gamma, beta):
    idx = unq_inv.astype(jnp.int32)
    y0, s_row, b_row = _mm_stats_pass(inputs, W, gamma.reshape(1, D),
                                      beta.reshape(1, D))
    zeros = jnp.zeros((N_SEG_PAD, D), jnp.float32)
    partials = _sc_segment_sum(y0, idx, zeros, s_row.reshape(D),
                               b_row.reshape(D))
    out = _combine(partials.reshape(NC, N_SEG_PAD, D))
    return out[:N_SEG]


# speedup vs baseline: 1.4450x; 1.0531x over previous
import functools

import jax
import jax.numpy as jnp
from jax import lax
from jax.experimental import pallas as pl
from jax.experimental.pallas import tpu as pltpu
from jax.experimental.pallas import tpu_sc as plsc

N_ROWS = 320000
D = 128
N_SEG = 10000
EPS = 0.001

BR1 = 12800
NB1 = N_ROWS // BR1


def _mm_stats_body(a_ref, w_ref, g_ref, bt_ref, y_ref, s_ref, b_ref,
                   cs_acc, ss_acc):
    i = pl.program_id(0)

    @pl.when(i == 0)
    def _():
        cs_acc[...] = jnp.zeros_like(cs_acc)
        ss_acc[...] = jnp.zeros_like(ss_acc)

    x = lax.dot_general(a_ref[...], w_ref[...], (((1,), (1,)), ((), ())),
                        preferred_element_type=jnp.float32)
    y_ref[...] = x
    cs_acc[...] += jnp.sum(x, axis=0, keepdims=True)
    ss_acc[...] += jnp.sum(x * x, axis=0, keepdims=True)

    @pl.when(i == NB1 - 1)
    def _():
        mean = cs_acc[...] / N_ROWS
        var = ss_acc[...] / N_ROWS - mean * mean
        s = g_ref[...] / jnp.sqrt(var + EPS)
        s_ref[...] = s
        b_ref[...] = bt_ref[...] - mean * s


def _mm_stats_pass(a, w, gamma_row, beta_row):
    return pl.pallas_call(
        _mm_stats_body,
        grid=(NB1,),
        in_specs=[
            pl.BlockSpec((BR1, D), lambda i: (i, 0)),
            pl.BlockSpec((D, D), lambda i: (0, 0)),
            pl.BlockSpec((1, D), lambda i: (0, 0)),
            pl.BlockSpec((1, D), lambda i: (0, 0)),
        ],
        out_specs=[
            pl.BlockSpec((BR1, D), lambda i: (i, 0)),
            pl.BlockSpec((1, D), lambda i: (0, 0)),
            pl.BlockSpec((1, D), lambda i: (0, 0)),
        ],
        out_shape=[
            jax.ShapeDtypeStruct((N_ROWS, D), jnp.float32),
            jax.ShapeDtypeStruct((1, D), jnp.float32),
            jax.ShapeDtypeStruct((1, D), jnp.float32),
        ],
        scratch_shapes=[
            pltpu.VMEM((1, D), jnp.float32),
            pltpu.VMEM((1, D), jnp.float32),
        ],
    )(a, w, gamma_row, beta_row)


_SC_INFO = plsc.get_sparse_core_info()
NC = _SC_INFO.num_cores
NS = _SC_INFO.num_subcores
NW = NC * NS
L = _SC_INFO.num_lanes
NG = D // L
ROWS_PER_W = N_ROWS // NW
WIN = 96
NWIN = ROWS_PER_W // WIN
TAIL = ROWS_PER_W - NWIN * WIN
N_SEG_PAD = 10240
SEG_PER_TILE = N_SEG_PAD // NS
NBUF = 3


def _sc_segment_sum(y, idx, zeros, s_row, b_row):
    mesh = plsc.VectorSubcoreMesh(core_axis_name="c", subcore_axis_name="s")

    scratch = (
        [pltpu.VMEM((WIN,), jnp.int32) for _ in range(NBUF)]
        + [pltpu.VMEM((WIN, D), jnp.float32) for _ in range(NBUF)]
        + [pltpu.VMEM((TAIL,), jnp.int32), pltpu.VMEM((TAIL, D), jnp.float32),
           pltpu.VMEM((D,), jnp.float32), pltpu.VMEM((D,), jnp.float32),
           pltpu.VMEM_SHARED((N_SEG_PAD, D), jnp.float32)]
        + [pltpu.SemaphoreType.DMA for _ in range(3 * NBUF + 1)]
    )

    @functools.partial(
        pl.kernel,
        out_type=jax.ShapeDtypeStruct((NC * N_SEG_PAD, D), jnp.float32),
        mesh=mesh,
        scratch_types=scratch,
    )
    def k(y_hbm, idx_hbm, zeros_hbm, s_hbm, b_hbm, out_hbm, *refs):
        idx_bufs = refs[0:NBUF]
        y_bufs = refs[NBUF:2 * NBUF]
        idx_t, y_t, s_v, b_v, acc = refs[2 * NBUF:2 * NBUF + 5]
        sems = refs[2 * NBUF + 5:]
        sem_i = sems[0:NBUF]
        sem_y = sems[NBUF:2 * NBUF]
        sem_s = sems[2 * NBUF:3 * NBUF]
        sem_misc = sems[3 * NBUF]

        c = lax.axis_index("c")
        s = lax.axis_index("s")
        wid = s * NC + c
        base0 = wid * ROWS_PER_W

        def start_gather(w, b):
            base = base0 + w * WIN
            pltpu.async_copy(idx_hbm.at[pl.ds(base, WIN)], idx_bufs[b], sem_i[b])
            pltpu.async_copy(y_hbm.at[pl.ds(base, WIN)], y_bufs[b], sem_y[b])

        def wait_gather(b):
            pltpu.make_async_copy(idx_hbm.at[pl.ds(0, WIN)], idx_bufs[b],
                                  sem_i[b]).wait()
            pltpu.make_async_copy(y_hbm.at[pl.ds(0, WIN)], y_bufs[b],
                                  sem_y[b]).wait()

        def start_scatter(b):
            pltpu.async_copy(y_bufs[b], acc.at[idx_bufs[b]], sem_s[b], add=True)

        def wait_scatter(b):
            pltpu.make_async_copy(y_bufs[b], acc.at[idx_bufs[b]],
                                  sem_s[b]).wait()

        pltpu.sync_copy(s_hbm, s_v)
        pltpu.sync_copy(b_hbm, b_v)
        s_regs = [s_v[pl.ds(j * L, L)] for j in range(NG)]
        b_regs = [b_v[pl.ds(j * L, L)] for j in range(NG)]

        def affine(b):
            yb = y_bufs[b]

            def row(r, carry):
                for j in range(NG):
                    v = yb[r, pl.ds(j * L, L)]
                    yb[r, pl.ds(j * L, L)] = jnp.maximum(
                        v * s_regs[j] + b_regs[j], 0.0)
                return carry

            lax.fori_loop(0, WIN, row, 0)

        pltpu.sync_copy(zeros_hbm.at[pl.ds(s * SEG_PER_TILE, SEG_PER_TILE)],
                        acc.at[pl.ds(s * SEG_PER_TILE, SEG_PER_TILE)])
        plsc.subcore_barrier()

        start_gather(0, 0)
        start_gather(1, 1)
        wait_gather(0)
        affine(0)
        start_scatter(0)
        start_gather(2, 2)

        def body3(i, carry):
            w0 = i * NBUF + 1
            for u in range(NBUF):
                w = w0 + u
                b = (1 + u) % NBUF
                bb = u
                wait_gather(b)
                affine(b)
                start_scatter(b)
                wait_scatter(bb)
                start_gather(w + 2, bb)
            return carry

        n_triples = (NWIN - 3) // NBUF
        lax.fori_loop(0, n_triples, body3, 0)
        for w in range(n_triples * NBUF + 1, NWIN):
            b = w % NBUF
            wait_gather(b)
            affine(b)
            start_scatter(b)
            if w + 2 < NWIN:
                bb = (w + 2) % NBUF
                wait_scatter(bb)
                start_gather(w + 2, bb)

        tbase = base0 + NWIN * WIN
        pltpu.sync_copy(idx_hbm.at[pl.ds(tbase, TAIL)], idx_t)
        pltpu.sync_copy(y_hbm.at[pl.ds(tbase, TAIL)], y_t)

        def trow(r, carry):
            for j in range(NG):
                v = y_t[r, pl.ds(j * L, L)]
                y_t[r, pl.ds(j * L, L)] = jnp.maximum(
                    v * s_regs[j] + b_regs[j], 0.0)
            return carry

        lax.fori_loop(0, TAIL, trow, 0)
        pltpu.sync_copy(y_t, acc.at[idx_t], add=True)

        for b in range(NBUF):
            wait_scatter(b)

        plsc.subcore_barrier()
        pltpu.sync_copy(
            acc.at[pl.ds(s * SEG_PER_TILE, SEG_PER_TILE)],
            out_hbm.at[pl.ds(c * N_SEG_PAD + s * SEG_PER_TILE, SEG_PER_TILE)])

    return k(y, idx, zeros, s_row, b_row)


BR3 = 1000
NB3 = N_SEG // BR3


def _combine_body(p_ref, o_ref):
    o_ref[...] = p_ref[0] + p_ref[1]


def _combine(p):
    return pl.pallas_call(
        _combine_body,
        grid=(NB3,),
        in_specs=[pl.BlockSpec((2, BR3, D), lambda i: (0, i, 0))],
        out_specs=pl.BlockSpec((BR3, D), lambda i: (i, 0)),
        out_shape=jax.ShapeDtypeStruct((N_SEG, D), jnp.float32),
    )(p)


def kernel(inputs, unq_inv, W, gamma, beta):
    idx = unq_inv.astype(jnp.int32)
    y0, s_row, b_row = _mm_stats_pass(inputs, W, gamma.reshape(1, D),
                                      beta.reshape(1, D))
    zeros = jnp.zeros((N_SEG_PAD, D), jnp.float32)
    partials = _sc_segment_sum(y0, idx, zeros, s_row.reshape(D),
                               b_row.reshape(D))
    return _combine(partials.reshape(NC, N_SEG_PAD, D))
